# Initial kernel scaffold; baseline (speedup 1.0000x reference)
import jax
import jax.numpy as jnp
from jax.experimental import pallas as pl


def kernel(x, edge_index, edge_weight, W1, b1, W2, b2, Wfc, bfc):
    raise NotImplementedError("write your pallas kernel here")


# speedup vs baseline: 10.7898x; 10.7898x over previous
import jax
import jax.numpy as jnp
from jax import lax
from jax.experimental import pallas as pl
from jax.experimental.pallas import tpu as pltpu
from jax.experimental.pallas import tpu_sc as plsc

N = 10000
E = 320000
DIN = 128
DH = 256
HALF = 128
NC, NS, L = 2, 16, 16
NW = NC * NS

CH = 128
NCK = E // CH
CPT = NCK // NS
XT = NCK % NS
CPW = NCK // NW
XW = NCK % NW

NP = 10240
NZT = 5
ZR = N // NZT
ZB = 40

DEG_CH = 80
DEG_NCH = E // (NS * DEG_CH)

BN = 1000
NB = N // BN


def _deg_body(dstb, ewb, out, dstv, valv, zb, deg_sh, sem):
    c = lax.axis_index("c")
    s = lax.axis_index("s")
    pltpu.sync_copy(dstb.at[s], dstv)
    pltpu.sync_copy(ewb.at[s], valv)

    def zloop(i, cc):
        zb[pl.ds(i * 16, 16)] = jnp.zeros((16,), jnp.float32)
        return cc
    lax.fori_loop(0, 128, zloop, 0)
    for i in range(NZT):
        @pl.when(s == i)
        def _():
            pltpu.sync_copy(zb, deg_sh.at[pl.ds(i * 2048, 2048)])
    plsc.subcore_barrier()

    half = DEG_NCH // 2

    def body(k, cc):
        kk = c * half + k
        pltpu.sync_copy(valv.at[kk], deg_sh.at[dstv.at[kk]], add=True)
        return cc
    lax.fori_loop(0, half, body, 0)
    plsc.subcore_barrier()

    for i in range(NZT):
        @pl.when(s == i)
        def _():
            pltpu.sync_copy(deg_sh.at[pl.ds(i * 2048, 2048)],
                            out.at[c, 0, pl.ds(i * 2048, 2048)])


def _sc_deg(dstb, ewb):
    mesh = plsc.VectorSubcoreMesh(core_axis_name="c", subcore_axis_name="s")
    f = pl.kernel(
        _deg_body,
        out_type=jax.ShapeDtypeStruct((NC, 1, NP), jnp.float32),
        mesh=mesh,
        scratch_types=[
            pltpu.VMEM((DEG_NCH, DEG_CH), jnp.int32),
            pltpu.VMEM((DEG_NCH, DEG_CH), jnp.float32),
            pltpu.VMEM((2048,), jnp.float32),
            pltpu.VMEM_SHARED((NP,), jnp.float32),
            pltpu.SemaphoreType.DMA,
        ],
    )
    return f(dstb, ewb)


def _norm_body(srcf, dstf, ewf, disf, nrm_out,
               sidx, didx, ewv, dSv, dDv, sem_i, sem_g):
    c = lax.axis_index("c")
    s = lax.axis_index("s")
    w = s * NC + c
    nchunks = CPW + jnp.where(w < XW, 1, 0)

    def start_idx(j):
        off = (j * NW + w) * CH
        pltpu.make_async_copy(srcf.at[pl.ds(off, CH)],
                              sidx.at[lax.rem(j, 2)], sem_i).start()
        pltpu.make_async_copy(dstf.at[pl.ds(off, CH)],
                              didx.at[lax.rem(j, 2)], sem_i).start()
        pltpu.make_async_copy(ewf.at[pl.ds(off, CH)],
                              ewv.at[lax.rem(j, 2)], sem_i).start()

    start_idx(0)

    def body(j, cc):
        slot = lax.rem(j, 2)
        off = (j * NW + w) * CH
        pltpu.make_async_copy(srcf.at[pl.ds(off, CH)], sidx.at[slot],
                              sem_i).wait()
        pltpu.make_async_copy(dstf.at[pl.ds(off, CH)], didx.at[slot],
                              sem_i).wait()
        pltpu.make_async_copy(ewf.at[pl.ds(off, CH)], ewv.at[slot],
                              sem_i).wait()

        @pl.when(j + 1 < nchunks)
        def _():
            start_idx(j + 1)

        a = pltpu.make_async_copy(disf.at[sidx.at[slot]], dSv, sem_g)
        b = pltpu.make_async_copy(disf.at[didx.at[slot]], dDv, sem_g)
        a.start()
        b.start()
        a.wait()
        b.wait()

        def gbody(g, c2):
            sl = pl.ds(g * 16, 16)
            dSv[sl] = dSv[sl] * ewv[slot, sl] * dDv[sl]
            return c2
        lax.fori_loop(0, CH // 16, gbody, 0)
        pltpu.sync_copy(dSv, nrm_out.at[pl.ds(off, CH)])
        return cc

    lax.fori_loop(0, nchunks, body, 0)


def _sc_norm(srcf, dstf, ewf, disf):
    mesh = plsc.VectorSubcoreMesh(core_axis_name="c", subcore_axis_name="s")
    f = pl.kernel(
        _norm_body,
        out_type=jax.ShapeDtypeStruct((E,), jnp.float32),
        mesh=mesh,
        scratch_types=[
            pltpu.VMEM((2, CH), jnp.int32),
            pltpu.VMEM((2, CH), jnp.int32),
            pltpu.VMEM((2, CH), jnp.float32),
            pltpu.VMEM((CH,), jnp.float32),
            pltpu.VMEM((CH,), jnp.float32),
            pltpu.SemaphoreType.DMA,
            pltpu.SemaphoreType.DMA,
        ],
    )
    return f(srcf, dstf, ewf, disf)


def _agg_body(srcf, dstf, nrmf, tab, S_out,
              sidx, didx, nrmv, gidxv, rowsv, zobuf, acc_sh, sem_i, sem_g):
    c = lax.axis_index("c")
    s = lax.axis_index("s")
    nchunks = CPT + jnp.where(s < XT, 1, 0)

    def zfill(r, cc):
        for j in range(HALF // 16):
            zobuf[r, pl.ds(j * 16, 16)] = jnp.zeros((16,), jnp.float32)
        return cc
    lax.fori_loop(0, ZB, zfill, 0)

    @pl.when(s < NZT)
    def _():
        def zloop(j, cc):
            pltpu.sync_copy(zobuf, acc_sh.at[pl.ds(s * ZR + j * ZB, ZB)])
            return cc
        lax.fori_loop(0, ZR // ZB, zloop, 0)

    def start_idx(j):
        off = (j * NS + s) * CH
        pltpu.make_async_copy(srcf.at[pl.ds(off, CH)],
                              sidx.at[lax.rem(j, 2)], sem_i).start()
        pltpu.make_async_copy(dstf.at[pl.ds(off, CH)],
                              didx.at[lax.rem(j, 2)], sem_i).start()
        pltpu.make_async_copy(nrmf.at[pl.ds(off, CH)],
                              nrmv.at[lax.rem(j, 2)], sem_i).start()

    start_idx(0)
    plsc.subcore_barrier()

    def body(j, cc):
        slot = lax.rem(j, 2)
        off = (j * NS + s) * CH
        pltpu.make_async_copy(srcf.at[pl.ds(off, CH)], sidx.at[slot],
                              sem_i).wait()
        pltpu.make_async_copy(dstf.at[pl.ds(off, CH)], didx.at[slot],
                              sem_i).wait()
        pltpu.make_async_copy(nrmf.at[pl.ds(off, CH)], nrmv.at[slot],
                              sem_i).wait()

        @pl.when(j + 1 < nchunks)
        def _():
            start_idx(j + 1)

        def gi(g, c2):
            sl = pl.ds(g * 16, 16)
            gidxv[sl] = sidx[slot, sl] + c * N
            return c2
        lax.fori_loop(0, CH // 16, gi, 0)

        cp = pltpu.make_async_copy(tab.at[gidxv], rowsv, sem_g)
        cp.start()
        cp.wait()

        def mult(g, c2):
            nv = nrmv[slot, pl.ds(g * 16, 16)]
            for i in range(16):
                nrm = nv[i]
                for jj in range(HALF // 16):
                    sl = pl.ds(jj * 16, 16)
                    rowsv[g * 16 + i, sl] = rowsv[g * 16 + i, sl] * nrm
            return c2
        lax.fori_loop(0, CH // 16, mult, 0)

        pltpu.sync_copy(rowsv, acc_sh.at[didx.at[slot]], add=True)
        return cc

    lax.fori_loop(0, nchunks, body, 0)
    plsc.subcore_barrier()

    @pl.when(s < NZT)
    def _():
        def wloop(j, cc):
            pltpu.sync_copy(acc_sh.at[pl.ds(s * ZR + j * ZB, ZB)], zobuf)
            pltpu.sync_copy(zobuf, S_out.at[c, s, pl.ds(j * ZB, ZB)])
            return cc
        lax.fori_loop(0, ZR // ZB, wloop, 0)


def _sc_agg(srcf, dstf, nrmf, tab):
    mesh = plsc.VectorSubcoreMesh(core_axis_name="c", subcore_axis_name="s")
    f = pl.kernel(
        _agg_body,
        out_type=jax.ShapeDtypeStruct((NC, NZT, ZR, HALF), jnp.float32),
        mesh=mesh,
        scratch_types=[
            pltpu.VMEM((2, CH), jnp.int32),
            pltpu.VMEM((2, CH), jnp.int32),
            pltpu.VMEM((2, CH), jnp.float32),
            pltpu.VMEM((CH,), jnp.int32),
            pltpu.VMEM((CH, HALF), jnp.float32),
            pltpu.VMEM((ZB, HALF), jnp.float32),
            pltpu.VMEM_SHARED((N, HALF), jnp.float32),
            pltpu.SemaphoreType.DMA,
            pltpu.SemaphoreType.DMA,
        ],
    )
    return f(srcf, dstf, nrmf, tab)


def _dis_body(deg_ref, out_ref):
    d = deg_ref[0, :] + deg_ref[1, :] + 1.0
    out_ref[...] = lax.rsqrt(d).reshape(1, N)


def _tc_dis(deg_p):
    return pl.pallas_call(
        _dis_body,
        out_shape=jax.ShapeDtypeStruct((1, N), jnp.float32),
    )(deg_p)


def _mm1_body(x_ref, w_ref, out_ref):
    out_ref[...] = jnp.dot(x_ref[...], w_ref[...],
                           preferred_element_type=jnp.float32)


def _tc_mm1(x, W1):
    return pl.pallas_call(
        _mm1_body,
        grid=(NC, NB),
        in_specs=[
            pl.BlockSpec((BN, DIN), lambda h, i: (i, 0)),
            pl.BlockSpec((DIN, HALF), lambda h, i: (0, h)),
        ],
        out_specs=pl.BlockSpec((BN, HALF), lambda h, i: (h * NB + i, 0)),
        out_shape=jax.ShapeDtypeStruct((NC * N, HALF), jnp.float32),
    )(x, W1)


def _mid_body(s0_ref, s1_ref, x0_ref, x1_ref, dis_ref, b_ref,
              w2_ref, out_ref):
    d2 = dis_ref[...] * dis_ref[...]
    z0 = jnp.maximum(s0_ref[...] + d2 * x0_ref[...] + b_ref[0:1, :], 0.0)
    z1 = jnp.maximum(s1_ref[...] + d2 * x1_ref[...] + b_ref[1:2, :], 0.0)
    z = jnp.concatenate([z0, z1], axis=1)
    out_ref[...] = jnp.dot(z, w2_ref[...], preferred_element_type=jnp.float32)


def _tc_mid(S1, xw1, dis_col, b1r, W2):
    half0 = pl.BlockSpec((BN, HALF), lambda h, i: (i, 0))
    half1 = pl.BlockSpec((BN, HALF), lambda h, i: (NB + i, 0))
    return pl.pallas_call(
        _mid_body,
        grid=(NC, NB),
        in_specs=[
            half0, half1, half0, half1,
            pl.BlockSpec((BN, 1), lambda h, i: (i, 0)),
            pl.BlockSpec((NC, HALF), lambda h, i: (0, 0)),
            pl.BlockSpec((DH, HALF), lambda h, i: (0, h)),
        ],
        out_specs=pl.BlockSpec((BN, HALF), lambda h, i: (h * NB + i, 0)),
        out_shape=jax.ShapeDtypeStruct((NC * N, HALF), jnp.float32),
    )(S1, S1, xw1, xw1, dis_col, b1r, W2)


def _fin_body(s0_ref, s1_ref, x0_ref, x1_ref, dis_ref, b_ref,
              wfc_ref, bfc_ref, out_ref, acc_ref):
    i = pl.program_id(0)

    @pl.when(i == 0)
    def _():
        acc_ref[...] = jnp.zeros_like(acc_ref)

    d2 = dis_ref[...] * dis_ref[...]
    z0 = jnp.maximum(s0_ref[...] + d2 * x0_ref[...] + b_ref[0:1, :], 0.0)
    z1 = jnp.maximum(s1_ref[...] + d2 * x1_ref[...] + b_ref[1:2, :], 0.0)
    z = jnp.concatenate([z0, z1], axis=1)
    acc_ref[...] += jnp.sum(z, axis=0, keepdims=True)

    @pl.when(i == NB - 1)
    def _():
        pooled = acc_ref[...] * (1.0 / N)
        out_ref[...] = (jnp.dot(pooled, wfc_ref[...],
                                preferred_element_type=jnp.float32)
                        + bfc_ref[...])


def _tc_fin(S2, zw2, dis_col, b2r, WfcT, bfcr):
    half0 = pl.BlockSpec((BN, HALF), lambda i: (i, 0))
    half1 = pl.BlockSpec((BN, HALF), lambda i: (NB + i, 0))
    return pl.pallas_call(
        _fin_body,
        grid=(NB,),
        in_specs=[
            half0, half1, half0, half1,
            pl.BlockSpec((BN, 1), lambda i: (i, 0)),
            pl.BlockSpec((NC, HALF), lambda i: (0, 0)),
            pl.BlockSpec((DH, 32), lambda i: (0, 0)),
            pl.BlockSpec((1, 32), lambda i: (0, 0)),
        ],
        out_specs=pl.BlockSpec((1, 32), lambda i: (0, 0)),
        out_shape=jax.ShapeDtypeStruct((1, 32), jnp.float32),
        scratch_shapes=[pltpu.VMEM((1, DH), jnp.float32)],
    )(S2, S2, zw2, zw2, dis_col, b2r, WfcT, bfcr)


def kernel(x, edge_index, edge_weight, W1, b1, W2, b2, Wfc, bfc):
    src = edge_index[0].astype(jnp.int32)
    dst = edge_index[1].astype(jnp.int32)
    dstb = dst.reshape(NS, DEG_NCH, DEG_CH)
    ewb = edge_weight.reshape(NS, DEG_NCH, DEG_CH)

    deg_p = _sc_deg(dstb, ewb).reshape(NC, NP)[:, :N]
    dis = _tc_dis(deg_p)
    dis1d = dis.reshape(N)
    dis_col = dis.reshape(N, 1)

    normf = _sc_norm(src, dst, edge_weight, dis1d)
    xw1 = _tc_mm1(x, W1)
    S1 = _sc_agg(src, dst, normf, xw1).reshape(NC * N, HALF)

    zw2 = _tc_mid(S1, xw1, dis_col, b1.reshape(NC, HALF), W2)
    S2 = _sc_agg(src, dst, normf, zw2).reshape(NC * N, HALF)

    out = _tc_fin(S2, zw2, dis_col, b2.reshape(NC, HALF), Wfc.T,
                  bfc.reshape(1, 32))
    return out.reshape(32)
